# Initial kernel scaffold; baseline (speedup 1.0000x reference)
import jax
import jax.numpy as jnp
from jax.experimental import pallas as pl


def kernel(input_spikes, weight):
    raise NotImplementedError("write your pallas kernel here")


# speedup vs baseline: 3.7204x; 3.7204x over previous
import functools

import jax
import jax.numpy as jnp
from jax import lax
from jax.experimental import pallas as pl
from jax.experimental.pallas import tpu as pltpu
from jax.experimental.pallas import tpu_sc as plsc

STEP = 16
LEAK = 32
KSIZE = STEP + LEAK
PAD_T = 32
FODEP = KSIZE
THETA = 2.7
B = 8
C = 8
T_IN = 64
XY = 23
N = XY * XY
NPAD = 544
T_OUT = 81
ROWS = T_OUT * C
CDIM = T_IN * 9
NCHUNK = NPAD // 16


def _weight_kernel(weight):
    t = jnp.arange(KSIZE, dtype=weight.dtype)
    t = jnp.broadcast_to(t, weight.shape + (KSIZE,))
    t_spike = t / STEP
    t_leak = -(t - weight[..., None] * STEP) / LEAK + weight[..., None]
    k = jnp.maximum(0.0, jnp.minimum(t_spike, t_leak))
    k = jnp.flip(k, -1)
    return jnp.transpose(k[:, 0], (0, 3, 1, 2))


def _build_m2(weight):
    wk = _weight_kernel(weight)
    tau = jnp.arange(T_OUT)[:, None]
    t = jnp.arange(T_IN)[None, :]
    dt = t - tau + PAD_T
    valid = (dt >= 0) & (dt < KSIZE)
    dtc = jnp.clip(dt, 0, KSIZE - 1)
    m = wk[:, dtc]
    m = jnp.where(valid[None, :, :, None, None], m, 0.0)
    m = jnp.transpose(m, (1, 0, 2, 3, 4))
    return m.reshape(ROWS, CDIM).astype(jnp.float32)


def _build_patches(input_spikes):
    x = jnp.transpose(input_spikes[:, 0], (0, 3, 1, 2))
    taps = [x[:, :, kx:kx + 45:2, ky:ky + 45:2]
            for kx in range(3) for ky in range(3)]
    p = jnp.stack(taps, axis=2)
    p = p.reshape(B, CDIM, N)
    return jnp.pad(p, ((0, 0), (0, 0), (0, NPAD - N)))


def _conv_body(m_ref, p_ref, o_ref):
    o_ref[0] = jnp.dot(m_ref[...], p_ref[0],
                       preferred_element_type=jnp.float32)


def _conv(m2, patches):
    return pl.pallas_call(
        _conv_body,
        grid=(B,),
        in_specs=[
            pl.BlockSpec((ROWS, CDIM), lambda b: (0, 0)),
            pl.BlockSpec((1, CDIM, NPAD), lambda b: (b, 0, 0)),
        ],
        out_specs=pl.BlockSpec((1, ROWS, NPAD), lambda b: (b, 0, 0)),
        out_shape=jax.ShapeDtypeStruct((B, ROWS, NPAD), jnp.float32),
    )(m2, patches)


def _wta_body(pot_hbm, win_hbm, potbuf, freeref, winbuf, sem0):
    wid = lax.axis_index("s") * 2 + lax.axis_index("c")

    @pl.when(wid < B)
    def _():
        idx0 = lax.iota(jnp.int32, 16)
        zero16 = jnp.zeros((16,), jnp.int32)

        def zf(j, carry):
            freeref[pl.ds(j * 16, 16)] = zero16
            return carry

        lax.fori_loop(0, NCHUNK, zf, 0)
        base_row = wid * ROWS

        def tau_body(tau, carry):
            pltpu.async_copy(
                pot_hbm.at[pl.ds(base_row + tau * C, C)],
                potbuf.at[pl.ds(0, C)], sem0).wait()

            def amax(j, st):
                bvs, bis = st
                base = j * 16
                ft = freeref[pl.ds(base, 16)]
                m = ft <= tau
                idxv = idx0 + base
                nbv, nbi = [], []
                for ch in range(C):
                    v = potbuf[ch, pl.ds(base, 16)]
                    mv = jnp.where(m, v, jnp.float32(0.0))
                    g = mv > bvs[ch]
                    nbv.append(jnp.where(g, mv, bvs[ch]))
                    nbi.append(jnp.where(g, idxv, bis[ch]))
                return (tuple(nbv), tuple(nbi))

            init = (tuple(jnp.full((16,), -1.0, jnp.float32) for _ in range(C)),
                    tuple(jnp.zeros((16,), jnp.int32) for _ in range(C)))
            bvs, bis = lax.fori_loop(0, NCHUNK, amax, init)

            win_vec = zero16
            smask = zero16
            for ch in range(C):
                mx = jnp.max(bvs[ch])
                cand = jnp.where(bvs[ch] == mx, bis[ch], jnp.int32(1 << 30))
                w = jnp.min(cand)
                s = (mx > THETA).astype(jnp.int32)
                win_vec = jnp.where(idx0 == ch, w, win_vec)
                win_vec = jnp.where(idx0 == C + ch, s, win_vec)
                smask = jnp.where(idx0 == ch, s, smask)
            winbuf[tau, :] = win_vec
            vals = zero16 + (tau + FODEP)
            plsc.store_scatter(freeref, [win_vec], vals, mask=smask > 0)
            return carry

        lax.fori_loop(0, T_OUT, tau_body, 0)
        pltpu.sync_copy(winbuf, win_hbm.at[wid])


def _wta(pot):
    info = plsc.get_sparse_core_info()
    mesh = plsc.VectorSubcoreMesh(
        core_axis_name="c", subcore_axis_name="s",
        num_cores=info.num_cores, num_subcores=info.num_subcores)
    return pl.kernel(
        _wta_body,
        out_type=jax.ShapeDtypeStruct((B, T_OUT, 16), jnp.int32),
        mesh=mesh,
        compiler_params=pltpu.CompilerParams(needs_layout_passes=False),
        scratch_types=[
            pltpu.VMEM((C, NPAD), jnp.float32),
            pltpu.VMEM((NPAD,), jnp.int32),
            pltpu.VMEM((T_OUT, 16), jnp.int32),
            pltpu.SemaphoreType.DMA,
        ],
    )(pot.reshape(B * ROWS, NPAD))


def _expand_body(win_ref, o_ref):
    n_iota = lax.broadcasted_iota(jnp.int32, (T_OUT, N), 1)
    for ch in range(C):
        w = win_ref[0, ch, :]
        s = win_ref[0, C + ch, :]
        hit = (n_iota == w[:, None]) & (s[:, None] > 0)
        o_ref[0, ch] = hit.astype(jnp.float32)


def _expand(win):
    win_t = jnp.transpose(win, (0, 2, 1))
    return pl.pallas_call(
        _expand_body,
        grid=(B,),
        in_specs=[pl.BlockSpec((1, 16, T_OUT), lambda b: (b, 0, 0))],
        out_specs=pl.BlockSpec((1, C, T_OUT, N), lambda b: (b, 0, 0, 0)),
        out_shape=jax.ShapeDtypeStruct((B, C, T_OUT, N), jnp.float32),
    )(win_t)


def kernel(input_spikes, weight):
    m2 = _build_m2(weight)
    patches = _build_patches(input_spikes)
    pot = _conv(m2, patches)
    win = _wta(pot)
    out = _expand(win)
    return out.reshape(B, C, XY, XY, T_OUT)


# speedup vs baseline: 20.1220x; 5.4086x over previous
import functools

import jax
import jax.numpy as jnp
from jax import lax
from jax.experimental import pallas as pl
from jax.experimental.pallas import tpu as pltpu
from jax.experimental.pallas import tpu_sc as plsc

STEP = 16
LEAK = 32
KSIZE = STEP + LEAK
PAD_T = 32
FODEP = KSIZE
THETA = 2.7
B = 8
C = 8
T_IN = 64
XY = 23
N = XY * XY
GRID = 25
NPAD = 640
NSRC = 672
T_OUT = 81
ROWS = T_OUT * C
CDIM = T_IN * 9
NCHUNK = NPAD // 16
TGRP = 9
NGRP = T_OUT // TGRP
BIGFREE = 1 << 20

TAPS = [(kx % 2, ky % 2, (kx // 2) * GRID + (ky // 2))
        for kx in range(3) for ky in range(3)]


def _weight_kernel(weight):
    t = jnp.arange(KSIZE, dtype=weight.dtype)
    t = jnp.broadcast_to(t, weight.shape + (KSIZE,))
    t_spike = t / STEP
    t_leak = -(t - weight[..., None] * STEP) / LEAK + weight[..., None]
    k = jnp.maximum(0.0, jnp.minimum(t_spike, t_leak))
    k = jnp.flip(k, -1)
    return jnp.transpose(k[:, 0], (0, 3, 1, 2))


def _build_m2(weight):
    wk = _weight_kernel(weight)
    tau = jnp.arange(T_OUT)[:, None]
    t = jnp.arange(T_IN)[None, :]
    dt = t - tau + PAD_T
    valid = (dt >= 0) & (dt < KSIZE)
    dtc = jnp.clip(dt, 0, KSIZE - 1)
    m = wk[:, dtc]
    m = jnp.where(valid[None, :, :, None, None], m, 0.0)
    m = jnp.transpose(m, (1, 0, 3, 4, 2))
    return m.reshape(ROWS, CDIM).astype(jnp.float32)


def _build_planes(input_spikes):
    x = jnp.pad(input_spikes[:, 0], ((0, 0), (0, 2), (0, 2), (0, 0)))
    x = x.reshape(B, GRID, 2, GRID, 2, T_IN)
    x = jnp.transpose(x, (0, 2, 4, 5, 1, 3))
    x = x.reshape(B, 2, 2, T_IN, GRID * GRID)
    return jnp.pad(x, ((0, 0), (0, 0), (0, 0), (0, 0), (0, NSRC - GRID * GRID)))


def _conv_body(m_ref, x_ref, o_ref, p_scr):
    for k, (px, py, sh) in enumerate(TAPS):
        p_scr[k * T_IN:(k + 1) * T_IN, :] = x_ref[0, px, py, :, pl.ds(sh, NPAD)]
    o_ref[0] = jnp.dot(m_ref[...], p_scr[...],
                       preferred_element_type=jnp.float32)


def _conv(m2, planes):
    return pl.pallas_call(
        _conv_body,
        grid=(B,),
        in_specs=[
            pl.BlockSpec((ROWS, CDIM), lambda b: (0, 0)),
            pl.BlockSpec((1, 2, 2, T_IN, NSRC), lambda b: (b, 0, 0, 0, 0)),
        ],
        out_specs=pl.BlockSpec((1, ROWS, NPAD), lambda b: (b, 0, 0)),
        out_shape=jax.ShapeDtypeStruct((B, ROWS, NPAD), jnp.float32),
        scratch_shapes=[pltpu.VMEM((CDIM, NPAD), jnp.float32)],
    )(m2, planes)


def _wta_body(pot_hbm, win_hbm, potbuf, freeref, winbuf, sem0, sem1):
    wid = lax.axis_index("s") * 2 + lax.axis_index("c")

    @pl.when(wid < B)
    def _():
        idx0 = lax.iota(jnp.int32, 16)
        zero16 = jnp.zeros((16,), jnp.int32)

        def zf(j, carry):
            nv = idx0 + j * 16
            xq = nv // GRID
            yr = nv - xq * GRID
            ok = (xq < XY) & (yr < XY)
            freeref[pl.ds(j * 16, 16)] = jnp.where(ok, 0, BIGFREE)
            return carry

        lax.fori_loop(0, NCHUNK, zf, 0)
        base_row = wid * ROWS
        rows_per_grp = TGRP * C

        def compute_tau(tau, buf_base):
            row0 = buf_base + (tau % TGRP) * C

            def amax(j, st):
                bvs, bis = st
                base = j * 16
                ft = freeref[pl.ds(base, 16)]
                m = ft <= tau
                idxv = idx0 + base
                nbv, nbi = [], []
                for ch in range(C):
                    v = potbuf[row0 + ch, pl.ds(base, 16)]
                    mv = jnp.where(m, v, jnp.float32(0.0))
                    g = mv > bvs[ch]
                    nbv.append(jnp.where(g, mv, bvs[ch]))
                    nbi.append(jnp.where(g, idxv, bis[ch]))
                return (tuple(nbv), tuple(nbi))

            init = (tuple(jnp.full((16,), -1.0, jnp.float32) for _ in range(C)),
                    tuple(jnp.zeros((16,), jnp.int32) for _ in range(C)))
            bvs, bis = lax.fori_loop(0, NCHUNK, amax, init)

            win_vec = zero16
            smask = zero16
            for ch in range(C):
                mx = jnp.max(bvs[ch])
                cand = jnp.where(bvs[ch] == mx, bis[ch], jnp.int32(1 << 30))
                w = jnp.min(cand)
                s = (mx > THETA).astype(jnp.int32)
                win_vec = jnp.where(idx0 == ch, w, win_vec)
                win_vec = jnp.where(idx0 == C + ch, s, win_vec)
                smask = jnp.where(idx0 == ch, s, smask)
            winbuf[tau, :] = win_vec
            vals = zero16 + (tau + FODEP)
            plsc.store_scatter(freeref, [win_vec], vals, mask=smask > 0)

        sems = (sem0, sem1)

        def issue(g):
            half = (g % 2) * rows_per_grp
            return pltpu.async_copy(
                pot_hbm.at[pl.ds(base_row + g * rows_per_grp, rows_per_grp)],
                potbuf.at[pl.ds(half, rows_per_grp)], sems[g % 2])

        pending = issue(0)
        for g in range(NGRP):
            pending.wait()
            if g + 1 < NGRP:
                pending = issue(g + 1)
            buf_base = (g % 2) * rows_per_grp

            def grp_body(tl, carry):
                compute_tau(g * TGRP + tl, buf_base)
                return carry

            lax.fori_loop(0, TGRP, grp_body, 0)

        pltpu.sync_copy(winbuf, win_hbm.at[wid])


def _wta(pot):
    info = plsc.get_sparse_core_info()
    mesh = plsc.VectorSubcoreMesh(
        core_axis_name="c", subcore_axis_name="s",
        num_cores=info.num_cores, num_subcores=info.num_subcores)
    return pl.kernel(
        _wta_body,
        out_type=jax.ShapeDtypeStruct((B, T_OUT, 16), jnp.int32),
        mesh=mesh,
        compiler_params=pltpu.CompilerParams(needs_layout_passes=False),
        scratch_types=[
            pltpu.VMEM((2 * TGRP * C, NPAD), jnp.float32),
            pltpu.VMEM((NPAD,), jnp.int32),
            pltpu.VMEM((T_OUT, 16), jnp.int32),
            pltpu.SemaphoreType.DMA,
            pltpu.SemaphoreType.DMA,
        ],
    )(pot.reshape(B * ROWS, NPAD))


def _expand_body(win_ref, o_ref):
    n_iota = lax.broadcasted_iota(jnp.int32, (T_OUT, N), 1)
    np_iota = n_iota + 2 * (n_iota // XY)
    for ch in range(C):
        w = win_ref[0, ch, :]
        s = win_ref[0, C + ch, :]
        hit = (np_iota == w[:, None]) & (s[:, None] > 0)
        o_ref[0, ch] = hit.astype(jnp.float32)


def _expand(win):
    win_t = jnp.transpose(win, (0, 2, 1))
    return pl.pallas_call(
        _expand_body,
        grid=(B,),
        in_specs=[pl.BlockSpec((1, 16, T_OUT), lambda b: (b, 0, 0))],
        out_specs=pl.BlockSpec((1, C, T_OUT, N), lambda b: (b, 0, 0, 0)),
        out_shape=jax.ShapeDtypeStruct((B, C, T_OUT, N), jnp.float32),
    )(win_t)


def kernel(input_spikes, weight):
    m2 = _build_m2(weight)
    planes = _build_planes(input_spikes)
    pot = _conv(m2, planes)
    win = _wta(pot)
    out = _expand(win)
    return out.reshape(B, C, XY, XY, T_OUT)


# speedup vs baseline: 26.8113x; 1.3324x over previous
import functools

import jax
import jax.numpy as jnp
from jax import lax
from jax.experimental import pallas as pl
from jax.experimental.pallas import tpu as pltpu
from jax.experimental.pallas import tpu_sc as plsc

STEP = 16
LEAK = 32
KSIZE = STEP + LEAK
PAD_T = 32
FODEP = KSIZE
THETA = 2.7
B = 8
C = 8
T_IN = 64
XY = 23
N = XY * XY
GRID = 25
NPAD = 640
NSRC = 672
T_OUT = 81
ROWS = T_OUT * C
CDIM = T_IN * 9
NCHUNK = NPAD // 16
TGRP = 9
NGRP = T_OUT // TGRP
BIGFREE = 1 << 20

TAPS = [(kx % 2, ky % 2, (kx // 2) * GRID + (ky // 2))
        for kx in range(3) for ky in range(3)]


def _build_m2(weight):
    w = weight[:, 0].astype(jnp.float32)
    tau = jnp.arange(T_OUT, dtype=jnp.int32)
    t = jnp.arange(T_IN, dtype=jnp.int32)
    u_i = 15 + tau[:, None] - t[None, :]
    valid = (u_i >= 0) & (u_i < KSIZE)
    u = u_i.astype(jnp.float32)[:, None, None, None, :]
    wb = w[None, :, :, :, None]
    t_spike = u / STEP
    t_leak = -(u - wb * STEP) / LEAK + wb
    m = jnp.maximum(0.0, jnp.minimum(t_spike, t_leak))
    m = jnp.where(valid[:, None, None, None, :], m, 0.0)
    return m.reshape(ROWS, CDIM)


def _build_planes(input_spikes):
    x = jnp.pad(input_spikes[:, 0], ((0, 0), (0, 2), (0, 2), (0, 0)))
    x = x.reshape(B, GRID, 2, GRID, 2, T_IN)
    x = jnp.transpose(x, (0, 2, 4, 5, 1, 3))
    x = x.reshape(B, 2, 2, T_IN, GRID * GRID)
    return jnp.pad(x, ((0, 0), (0, 0), (0, 0), (0, 0), (0, NSRC - GRID * GRID)))


def _conv_body(m_ref, x_ref, o_ref, p_scr):
    for k, (px, py, sh) in enumerate(TAPS):
        p_scr[k * T_IN:(k + 1) * T_IN, :] = x_ref[0, px, py, :, pl.ds(sh, NPAD)]
    o_ref[0] = jnp.dot(m_ref[...], p_scr[...],
                       preferred_element_type=jnp.float32)


def _conv(m2, planes):
    return pl.pallas_call(
        _conv_body,
        grid=(B,),
        in_specs=[
            pl.BlockSpec((ROWS, CDIM), lambda b: (0, 0)),
            pl.BlockSpec((1, 2, 2, T_IN, NSRC), lambda b: (b, 0, 0, 0, 0)),
        ],
        out_specs=pl.BlockSpec((1, ROWS, NPAD), lambda b: (b, 0, 0)),
        out_shape=jax.ShapeDtypeStruct((B, ROWS, NPAD), jnp.float32),
        scratch_shapes=[pltpu.VMEM((CDIM, NPAD), jnp.float32)],
    )(m2, planes)


def _wta_body(pot_hbm, win_hbm, potbuf, freeref, winbuf, sem0, sem1):
    wid = lax.axis_index("s") * 2 + lax.axis_index("c")

    @pl.when(wid < B)
    def _():
        idx0 = lax.iota(jnp.int32, 16)
        zero16 = jnp.zeros((16,), jnp.int32)

        def zf(j, carry):
            nv = idx0 + j * 16
            xq = nv // GRID
            yr = nv - xq * GRID
            ok = (xq < XY) & (yr < XY)
            freeref[pl.ds(j * 16, 16)] = jnp.where(ok, 0, BIGFREE)
            return carry

        lax.fori_loop(0, NCHUNK, zf, 0)
        base_row = wid * ROWS
        rows_per_grp = TGRP * C

        def compute_tau(tau, buf_base):
            row0 = buf_base + (tau % TGRP) * C

            def amax(j, st):
                bvs, bis = st
                base = j * 16
                ft = freeref[pl.ds(base, 16)]
                m = ft <= tau
                idxv = idx0 + base
                nbv, nbi = [], []
                for ch in range(C):
                    v = potbuf[row0 + ch, pl.ds(base, 16)]
                    mv = jnp.where(m, v, jnp.float32(0.0))
                    g = mv > bvs[ch]
                    nbv.append(jnp.where(g, mv, bvs[ch]))
                    nbi.append(jnp.where(g, idxv, bis[ch]))
                return (tuple(nbv), tuple(nbi))

            init = (tuple(jnp.full((16,), -1.0, jnp.float32) for _ in range(C)),
                    tuple(jnp.zeros((16,), jnp.int32) for _ in range(C)))
            bvs, bis = lax.fori_loop(0, NCHUNK, amax, init)

            win_vec = zero16
            smask = zero16
            for ch in range(C):
                mx = jnp.max(bvs[ch])
                cand = jnp.where(bvs[ch] == mx, bis[ch], jnp.int32(1 << 30))
                w = jnp.min(cand)
                s = (mx > THETA).astype(jnp.int32)
                win_vec = jnp.where(idx0 == ch, w, win_vec)
                win_vec = jnp.where(idx0 == C + ch, s, win_vec)
                smask = jnp.where(idx0 == ch, s, smask)
            winbuf[tau, :] = win_vec
            vals = zero16 + (tau + FODEP)
            plsc.store_scatter(freeref, [win_vec], vals, mask=smask > 0)

        sems = (sem0, sem1)

        def issue(g):
            half = (g % 2) * rows_per_grp
            return pltpu.async_copy(
                pot_hbm.at[pl.ds(base_row + g * rows_per_grp, rows_per_grp)],
                potbuf.at[pl.ds(half, rows_per_grp)], sems[g % 2])

        pending = issue(0)
        for g in range(NGRP):
            pending.wait()
            if g + 1 < NGRP:
                pending = issue(g + 1)
            buf_base = (g % 2) * rows_per_grp

            def grp_body(tl, carry):
                compute_tau(g * TGRP + tl, buf_base)
                return carry

            lax.fori_loop(0, TGRP, grp_body, 0)

        pltpu.sync_copy(winbuf, win_hbm.at[wid])


def _wta(pot):
    info = plsc.get_sparse_core_info()
    mesh = plsc.VectorSubcoreMesh(
        core_axis_name="c", subcore_axis_name="s",
        num_cores=info.num_cores, num_subcores=info.num_subcores)
    return pl.kernel(
        _wta_body,
        out_type=jax.ShapeDtypeStruct((B, T_OUT, 16), jnp.int32),
        mesh=mesh,
        compiler_params=pltpu.CompilerParams(needs_layout_passes=False),
        scratch_types=[
            pltpu.VMEM((2 * TGRP * C, NPAD), jnp.float32),
            pltpu.VMEM((NPAD,), jnp.int32),
            pltpu.VMEM((T_OUT, 16), jnp.int32),
            pltpu.SemaphoreType.DMA,
            pltpu.SemaphoreType.DMA,
        ],
    )(pot.reshape(B * ROWS, NPAD))


def _expand_body(win_ref, o_ref):
    n_iota = lax.broadcasted_iota(jnp.int32, (T_OUT, N), 1)
    np_iota = n_iota + 2 * (n_iota // XY)
    for ch in range(C):
        w = win_ref[0, ch, :]
        s = win_ref[0, C + ch, :]
        hit = (np_iota == w[:, None]) & (s[:, None] > 0)
        o_ref[0, ch] = hit.astype(jnp.float32)


def _expand(win):
    win_t = jnp.transpose(win, (0, 2, 1))
    return pl.pallas_call(
        _expand_body,
        grid=(B,),
        in_specs=[pl.BlockSpec((1, 16, T_OUT), lambda b: (b, 0, 0))],
        out_specs=pl.BlockSpec((1, C, T_OUT, N), lambda b: (b, 0, 0, 0)),
        out_shape=jax.ShapeDtypeStruct((B, C, T_OUT, N), jnp.float32),
    )(win_t)


def kernel(input_spikes, weight):
    m2 = _build_m2(weight)
    planes = _build_planes(input_spikes)
    pot = _conv(m2, planes)
    win = _wta(pot)
    out = _expand(win)
    return out.reshape(B, C, XY, XY, T_OUT)


# speedup vs baseline: 29.0660x; 1.0841x over previous
import functools

import jax
import jax.numpy as jnp
from jax import lax
from jax.experimental import pallas as pl
from jax.experimental.pallas import tpu as pltpu
from jax.experimental.pallas import tpu_sc as plsc

STEP = 16
LEAK = 32
KSIZE = STEP + LEAK
PAD_T = 32
FODEP = KSIZE
THETA = 2.7
B = 8
C = 8
T_IN = 64
XY = 23
N = XY * XY
GRID = 25
NPAD = 640
NSRC = 672
T_OUT = 81
ROWS = T_OUT * C
CDIM = T_IN * 9
NCHUNK = NPAD // 16
TGRP = 9
NGRP = T_OUT // TGRP
BIGFREE = 1 << 20

TAPS = [(kx % 2, ky % 2, (kx // 2) * GRID + (ky // 2))
        for kx in range(3) for ky in range(3)]


def _build_m2(weight):
    w = weight[:, 0].astype(jnp.float32)
    tau = jnp.arange(T_OUT, dtype=jnp.int32)
    t = jnp.arange(T_IN, dtype=jnp.int32)
    u_i = 15 + tau[:, None] - t[None, :]
    valid = (u_i >= 0) & (u_i < KSIZE)
    u = u_i.astype(jnp.float32)[:, None, None, None, :]
    wb = w[None, :, :, :, None]
    t_spike = u / STEP
    t_leak = -(u - wb * STEP) / LEAK + wb
    m = jnp.maximum(0.0, jnp.minimum(t_spike, t_leak))
    m = jnp.where(valid[:, None, None, None, :], m, 0.0)
    return m.reshape(ROWS, CDIM)


def _build_planes(input_spikes):
    x = jnp.pad(input_spikes[:, 0], ((0, 0), (0, 2), (0, 2), (0, 0)))
    x = x.reshape(B, GRID, 2, GRID, 2, T_IN)
    x = jnp.transpose(x, (0, 2, 4, 5, 1, 3))
    x = x.reshape(B, 2, 2, T_IN, GRID * GRID)
    return jnp.pad(x, ((0, 0), (0, 0), (0, 0), (0, 0), (0, NSRC - GRID * GRID)))


def _conv_body(m_ref, x_ref, o_ref, p_scr):
    for k, (px, py, sh) in enumerate(TAPS):
        p_scr[k * T_IN:(k + 1) * T_IN, :] = x_ref[0, px, py, :, pl.ds(sh, NPAD)]
    o_ref[0] = jnp.dot(m_ref[...], p_scr[...],
                       preferred_element_type=jnp.float32)


def _conv(m2, planes):
    return pl.pallas_call(
        _conv_body,
        grid=(B,),
        in_specs=[
            pl.BlockSpec((ROWS, CDIM), lambda b: (0, 0)),
            pl.BlockSpec((1, 2, 2, T_IN, NSRC), lambda b: (b, 0, 0, 0, 0)),
        ],
        out_specs=pl.BlockSpec((1, ROWS, NPAD), lambda b: (b, 0, 0)),
        out_shape=jax.ShapeDtypeStruct((B, ROWS, NPAD), jnp.float32),
        scratch_shapes=[pltpu.VMEM((CDIM, NPAD), jnp.float32)],
    )(m2, planes)


def _wta_body(pot_hbm, win_hbm, potbuf, freeref, winbuf, sem0, sem1):
    wid = lax.axis_index("s") * 2 + lax.axis_index("c")

    @pl.when(wid < B)
    def _():
        idx0 = lax.iota(jnp.int32, 16)
        zero16 = jnp.zeros((16,), jnp.int32)

        def zf(j, carry):
            nv = idx0 + j * 16
            xq = nv // GRID
            yr = nv - xq * GRID
            ok = (xq < XY) & (yr < XY)
            freeref[pl.ds(j * 16, 16)] = jnp.where(ok, 0, BIGFREE)
            return carry

        lax.fori_loop(0, NCHUNK, zf, 0)
        base_row = wid * ROWS
        rows_per_grp = TGRP * C

        def compute_tau(tau, buf_base):
            row0 = buf_base + (tau % TGRP) * C

            def amax(j, st):
                bvs, bis = st
                base = j * 16
                ft = freeref[pl.ds(base, 16)]
                m = ft <= tau
                idxv = idx0 + base
                nbv, nbi = [], []
                for ch in range(C):
                    v = potbuf[row0 + ch, pl.ds(base, 16)]
                    mv = jnp.where(m, v, jnp.float32(0.0))
                    g = mv > bvs[ch]
                    nbv.append(jnp.where(g, mv, bvs[ch]))
                    nbi.append(jnp.where(g, idxv, bis[ch]))
                return (tuple(nbv), tuple(nbi))

            init = (tuple(jnp.full((16,), -1.0, jnp.float32) for _ in range(C)),
                    tuple(jnp.zeros((16,), jnp.int32) for _ in range(C)))
            bvs, bis = lax.fori_loop(0, NCHUNK, amax, init)

            win_vec = zero16
            smask = zero16
            for ch in range(C):
                mx = jnp.max(bvs[ch])
                cand = jnp.where(bvs[ch] == mx, bis[ch], jnp.int32(1 << 30))
                w = jnp.min(cand)
                s = (mx > THETA).astype(jnp.int32)
                win_vec = jnp.where(idx0 == ch, w, win_vec)
                win_vec = jnp.where(idx0 == C + ch, s, win_vec)
                smask = jnp.where(idx0 == ch, s, smask)
            winbuf[tau, :] = win_vec
            vals = zero16 + (tau + FODEP)
            plsc.store_scatter(freeref, [win_vec], vals, mask=smask > 0)

        sems = (sem0, sem1)

        def issue(g):
            half = (g % 2) * rows_per_grp
            return pltpu.async_copy(
                pot_hbm.at[pl.ds(base_row + g * rows_per_grp, rows_per_grp)],
                potbuf.at[pl.ds(half, rows_per_grp)], sems[g % 2])

        pending = issue(0)
        for g in range(NGRP):
            pending.wait()
            if g + 1 < NGRP:
                pending = issue(g + 1)
            buf_base = (g % 2) * rows_per_grp

            def grp_body(tl, carry):
                compute_tau(g * TGRP + tl, buf_base)
                return carry

            lax.fori_loop(0, TGRP, grp_body, 0)

        pltpu.sync_copy(winbuf, win_hbm.at[wid])


def _wta(pot):
    info = plsc.get_sparse_core_info()
    mesh = plsc.VectorSubcoreMesh(
        core_axis_name="c", subcore_axis_name="s",
        num_cores=info.num_cores, num_subcores=info.num_subcores)
    return pl.kernel(
        _wta_body,
        out_type=jax.ShapeDtypeStruct((B, T_OUT, 16), jnp.int32),
        mesh=mesh,
        compiler_params=pltpu.CompilerParams(needs_layout_passes=False),
        scratch_types=[
            pltpu.VMEM((2 * TGRP * C, NPAD), jnp.float32),
            pltpu.VMEM((NPAD,), jnp.int32),
            pltpu.VMEM((T_OUT, 16), jnp.int32),
            pltpu.SemaphoreType.DMA,
            pltpu.SemaphoreType.DMA,
        ],
    )(pot.reshape(B * ROWS, NPAD))


import numpy as _np

XPAD = 24


def _expand_consts():
    n = _np.arange(N)
    q = (T_OUT * n) // N
    r = T_OUT * n - N * q
    row = XPAD * (n // XY) + (n % XY)
    t = _np.arange(T_OUT)
    m2 = (r[:, None] + t[None, :]) >= N
    n_old = r[:, None] + t[None, :] - N * m2
    npg = n_old + 2 * (n_old // XY)
    nr = XPAD * XY
    q1 = _np.zeros((nr, T_OUT), _np.float32)
    q2 = _np.zeros((nr, T_OUT), _np.float32)
    q1[row, q] = 1.0
    ok2 = q + 1 < T_OUT
    q2[row[ok2], q[ok2] + 1] = 1.0
    m2f = _np.zeros((nr, T_OUT), _np.float32)
    npgf = _np.full((nr, T_OUT), -1.0, _np.float32)
    m2f[row] = m2.astype(_np.float32)
    npgf[row] = npg.astype(_np.float32)
    return q1, q2, m2f, npgf


def _expand_body(q1_ref, q2_ref, m2_ref, npg_ref, win_ref, o_ref):
    winf = win_ref[0].astype(jnp.float32)
    g1 = jnp.dot(q1_ref[...], winf, preferred_element_type=jnp.float32)
    g2 = jnp.dot(q2_ref[...], winf)
    for x in range(XY):
        sl = slice(XPAD * x, XPAD * x + XY)
        m2x = m2_ref[sl, :]
        npgx = npg_ref[sl, :]
        for ch in range(C):
            wq = g1[sl, ch:ch + 1]
            wq1 = g2[sl, ch:ch + 1]
            sq = g1[sl, C + ch:C + ch + 1]
            sq1 = g2[sl, C + ch:C + ch + 1]
            val = jnp.where(m2x > 0.5, wq1, wq)
            spk = jnp.where(m2x > 0.5, sq1, sq)
            hit = (val == npgx) & (spk > 0.5)
            o_ref[0, ch, x] = hit.astype(jnp.float32)


def _expand(win):
    q1, q2, m2f, npgf = _expand_consts()
    nr = XPAD * XY
    cspec = pl.BlockSpec((nr, T_OUT), lambda b: (0, 0))
    return pl.pallas_call(
        _expand_body,
        grid=(B,),
        in_specs=[cspec, cspec, cspec, cspec,
                  pl.BlockSpec((1, T_OUT, 16), lambda b: (b, 0, 0))],
        out_specs=pl.BlockSpec((1, C, XY, XY, T_OUT),
                               lambda b: (b, 0, 0, 0, 0)),
        out_shape=jax.ShapeDtypeStruct((B, C, XY, XY, T_OUT), jnp.float32),
    )(jnp.asarray(q1), jnp.asarray(q2), jnp.asarray(m2f), jnp.asarray(npgf),
      win)


def kernel(input_spikes, weight):
    m2 = _build_m2(weight)
    planes = _build_planes(input_spikes)
    pot = _conv(m2, planes)
    win = _wta(pot)
    return _expand(win)
